# baseline (speedup 1.0000x reference)
import functools

import jax
import jax.numpy as jnp
from jax import lax
from jax.experimental import pallas as pl
from jax.experimental.pallas import tpu as pltpu
from jax.experimental.pallas import tpu_sc as plsc

B = 16384
D = 8
NC = 2
NS = 16
NW = NC * NS
BPW = B // NW
CH = 128
NCH = BPW // CH

@functools.lru_cache(maxsize=1)
def _make_sc_gather():
    mesh = plsc.VectorSubcoreMesh(core_axis_name="c", subcore_axis_name="s")

    @functools.partial(
        pl.kernel,
        mesh=mesh,
        compiler_params=pltpu.CompilerParams(use_tc_tiling_on_sc=False),
        out_type=[jax.ShapeDtypeStruct((B, D), jnp.float32) for _ in range(4)],
        scratch_types=[
            pltpu.VMEM((NCH, CH), jnp.int32),
            pltpu.VMEM((NCH, CH), jnp.int32),
            pltpu.VMEM((BPW, D), jnp.float32),
            pltpu.VMEM((BPW, D), jnp.float32),
            pltpu.VMEM((BPW, D), jnp.float32),
            pltpu.VMEM((BPW, D), jnp.float32),
            pltpu.SemaphoreType.DMA,
        ],
    )
    def _sc_gather(users_hbm, items_hbm, gu_hbm, gi_hbm, mu_hbm, mi_hbm,
                   o_gu, o_gi, o_mu, o_mi,
                   uidx, iidx, r_gu, r_gi, r_mu, r_mi, sem):
        wid = lax.axis_index("s") * NC + lax.axis_index("c")
        base = wid * BPW
        pltpu.sync_copy(users_hbm.at[pl.ds(wid * NCH, NCH)], uidx)
        pltpu.sync_copy(items_hbm.at[pl.ds(wid * NCH, NCH)], iidx)
        copies = []
        for j in range(NCH):
            sl = pl.ds(j * CH, CH)
            copies.append(pltpu.async_copy(gu_hbm.at[uidx.at[j]], r_gu.at[sl], sem))
            copies.append(pltpu.async_copy(gi_hbm.at[iidx.at[j]], r_gi.at[sl], sem))
            copies.append(pltpu.async_copy(mu_hbm.at[uidx.at[j]], r_mu.at[sl], sem))
            copies.append(pltpu.async_copy(mi_hbm.at[iidx.at[j]], r_mi.at[sl], sem))
        for c in copies:
            c.wait()
        out_sl = pl.ds(base, BPW)
        pltpu.sync_copy(r_gu, o_gu.at[out_sl])
        pltpu.sync_copy(r_gi, o_gi.at[out_sl])
        pltpu.sync_copy(r_mu, o_mu.at[out_sl])
        pltpu.sync_copy(r_mi, o_mi.at[out_sl])

    return _sc_gather


TC_BLK = 2048


def _tc_body(gu_ref, gi_ref, mu_ref, mi_ref,
             W0_ref, b0_ref, W1_ref, b1_ref, W2_ref, b2_ref, W3_ref, b3_ref,
             Wout_ref, bout_ref, out_ref):
    f32 = jnp.float32
    mu = mu_ref[...]
    mi = mi_ref[...]
    W0 = W0_ref[...]
    h = mu @ W0[:D, :] + mi @ W0[D:, :] + b0_ref[...]
    h = jnp.maximum(h, 0.0)
    h = jnp.maximum(h @ W1_ref[...] + b1_ref[...], 0.0)
    h = jnp.maximum(h @ W2_ref[...] + b2_ref[...], 0.0)
    h = jnp.maximum(h @ W3_ref[...] + b3_ref[...], 0.0)
    g = gu_ref[...] * gi_ref[...]
    Wout = Wout_ref[...]
    logit = h @ Wout[:D, :] + g @ Wout[D:, :] + bout_ref[...]
    out_ref[...] = jax.nn.sigmoid(logit).astype(f32)


def kernel(users, items, gmf_user_table, gmf_item_table, mlp_user_table,
           mlp_item_table, W0, b0, W1, b1, W2, b2, W3, b3, W_out, b_out):
    users_r = users.astype(jnp.int32).reshape(B // CH, CH)
    items_r = items.astype(jnp.int32).reshape(B // CH, CH)
    gu, gi, mu, mi = _make_sc_gather()(users_r, items_r, gmf_user_table,
                                       gmf_item_table, mlp_user_table,
                                       mlp_item_table)

    grid = B // TC_BLK
    data_spec = pl.BlockSpec((TC_BLK, D), lambda i: (i, 0))

    def wspec(shape):
        return pl.BlockSpec(shape, lambda i: tuple(0 for _ in shape))

    pred = pl.pallas_call(
        _tc_body,
        grid=(grid,),
        in_specs=[
            data_spec, data_spec, data_spec, data_spec,
            wspec(W0.shape), wspec((1, b0.shape[0])),
            wspec(W1.shape), wspec((1, b1.shape[0])),
            wspec(W2.shape), wspec((1, b2.shape[0])),
            wspec(W3.shape), wspec((1, b3.shape[0])),
            wspec(W_out.shape), wspec((1, 1)),
        ],
        out_specs=pl.BlockSpec((TC_BLK, 1), lambda i: (i, 0)),
        out_shape=jax.ShapeDtypeStruct((B, 1), jnp.float32),
    )(gu, gi, mu, mi,
      W0, b0.reshape(1, -1), W1, b1.reshape(1, -1), W2, b2.reshape(1, -1),
      W3, b3.reshape(1, -1), W_out, b_out.reshape(1, 1))
    return pred


# speedup vs baseline: 1.0093x; 1.0093x over previous
import functools

import jax
import jax.numpy as jnp
from jax import lax
from jax.experimental import pallas as pl
from jax.experimental.pallas import tpu as pltpu
from jax.experimental.pallas import tpu_sc as plsc

B = 16384
D = 8
NROWS = 1_000_000
RPB = 16
NBLK = NROWS // RPB
NC = 2
NS = 16
NW = NC * NS
BPW = B // NW
CH = 128
NCH = BPW // CH
LANES = 16


@functools.lru_cache(maxsize=1)
def _make_sc_gather():
    mesh = plsc.VectorSubcoreMesh(core_axis_name="c", subcore_axis_name="s")

    @functools.partial(
        pl.kernel,
        mesh=mesh,
        compiler_params=pltpu.CompilerParams(
            use_tc_tiling_on_sc=False, needs_layout_passes=False),
        out_type=[jax.ShapeDtypeStruct((D, B), jnp.float32) for _ in range(4)],
        scratch_types=[
            pltpu.VMEM((NCH, CH), jnp.int32),
            pltpu.VMEM((NCH, CH), jnp.int32),
            pltpu.VMEM((NCH, CH), jnp.int32),
            pltpu.VMEM((NCH, CH), jnp.int32),
            pltpu.VMEM((CH, RPB * D), jnp.float32),
            pltpu.VMEM((CH, RPB * D), jnp.float32),
            pltpu.VMEM((CH, RPB * D), jnp.float32),
            pltpu.VMEM((CH, RPB * D), jnp.float32),
            pltpu.VMEM((NCH, D, CH), jnp.float32),
            pltpu.VMEM((NCH, D, CH), jnp.float32),
            pltpu.VMEM((NCH, D, CH), jnp.float32),
            pltpu.VMEM((NCH, D, CH), jnp.float32),
            pltpu.SemaphoreType.DMA,
        ],
    )
    def _sc_gather(users_hbm, items_hbm, gu_hbm, gi_hbm, mu_hbm, mi_hbm,
                   o_gu, o_gi, o_mu, o_mi,
                   uidx, iidx, ublk, iblk, b_gu, b_gi, b_mu, b_mi,
                   t_gu, t_gi, t_mu, t_mi, sem):
        wid = lax.axis_index("s") * NC + lax.axis_index("c")
        base = wid * BPW
        pltpu.sync_copy(users_hbm.at[pl.ds(wid * NCH, NCH)], uidx)
        pltpu.sync_copy(items_hbm.at[pl.ds(wid * NCH, NCH)], iidx)
        for idx_ref, blk_ref in ((uidx, ublk), (iidx, iblk)):
            for k in range(NCH):
                for v in range(CH // LANES):
                    sl = pl.ds(v * LANES, LANES)
                    blk_ref[k, sl] = idx_ref[k, sl] >> 4
        lane = lax.iota(jnp.int32, LANES)
        for k in range(NCH):
            cps = [
                pltpu.async_copy(gu_hbm.at[ublk.at[k]], b_gu, sem),
                pltpu.async_copy(gi_hbm.at[iblk.at[k]], b_gi, sem),
                pltpu.async_copy(mu_hbm.at[ublk.at[k]], b_mu, sem),
                pltpu.async_copy(mi_hbm.at[iblk.at[k]], b_mi, sem),
            ]
            for cp in cps:
                cp.wait()
            for v in range(CH // LANES):
                sl = pl.ds(v * LANES, LANES)
                rows = lane + (v * LANES)
                ucol = ((uidx[k, sl] & (RPB - 1)) << 3)
                icol = ((iidx[k, sl] & (RPB - 1)) << 3)
                for c in range(D):
                    t_gu[k, c, sl] = plsc.load_gather(b_gu, [rows, ucol + c])
                    t_gi[k, c, sl] = plsc.load_gather(b_gi, [rows, icol + c])
                    t_mu[k, c, sl] = plsc.load_gather(b_mu, [rows, ucol + c])
                    t_mi[k, c, sl] = plsc.load_gather(b_mi, [rows, icol + c])
        for k in range(NCH):
            sl = pl.ds(base + k * CH, CH)
            pltpu.sync_copy(t_gu.at[k], o_gu.at[:, sl])
            pltpu.sync_copy(t_gi.at[k], o_gi.at[:, sl])
            pltpu.sync_copy(t_mu.at[k], o_mu.at[:, sl])
            pltpu.sync_copy(t_mi.at[k], o_mi.at[:, sl])

    return _sc_gather


TC_BLK = 2048


def _tc_body(guT_ref, giT_ref, muT_ref, miT_ref,
             W0uT_ref, W0iT_ref, b0_ref, W1T_ref, b1_ref, W2T_ref, b2_ref,
             W3T_ref, b3_ref, WohT_ref, WogT_ref, bo_ref, out_ref):
    muT = muT_ref[...]
    miT = miT_ref[...]
    h = W0uT_ref[...] @ muT + W0iT_ref[...] @ miT + b0_ref[...]
    h = jnp.maximum(h, 0.0)
    h = jnp.maximum(W1T_ref[...] @ h + b1_ref[...], 0.0)
    h = jnp.maximum(W2T_ref[...] @ h + b2_ref[...], 0.0)
    h = jnp.maximum(W3T_ref[...] @ h + b3_ref[...], 0.0)
    g = guT_ref[...] * giT_ref[...]
    logit = WohT_ref[...] @ h + WogT_ref[...] @ g + bo_ref[...]
    out_ref[...] = jax.nn.sigmoid(logit)


def kernel(users, items, gmf_user_table, gmf_item_table, mlp_user_table,
           mlp_item_table, W0, b0, W1, b1, W2, b2, W3, b3, W_out, b_out):
    users_r = users.astype(jnp.int32).reshape(B // CH, CH)
    items_r = items.astype(jnp.int32).reshape(B // CH, CH)
    guT, giT, muT, miT = _make_sc_gather()(
        users_r, items_r,
        gmf_user_table.reshape(NBLK, RPB * D),
        gmf_item_table.reshape(NBLK, RPB * D),
        mlp_user_table.reshape(NBLK, RPB * D),
        mlp_item_table.reshape(NBLK, RPB * D))

    grid = B // TC_BLK
    data_spec = pl.BlockSpec((D, TC_BLK), lambda i: (0, i))

    def wspec(shape):
        return pl.BlockSpec(shape, lambda i: tuple(0 for _ in shape))

    W0uT = W0[:D, :].T
    W0iT = W0[D:, :].T
    WohT = W_out[:D, :].T
    WogT = W_out[D:, :].T
    predT = pl.pallas_call(
        _tc_body,
        grid=(grid,),
        in_specs=[
            data_spec, data_spec, data_spec, data_spec,
            wspec(W0uT.shape), wspec(W0iT.shape), wspec((b0.shape[0], 1)),
            wspec(W1.T.shape), wspec((b1.shape[0], 1)),
            wspec(W2.T.shape), wspec((b2.shape[0], 1)),
            wspec(W3.T.shape), wspec((b3.shape[0], 1)),
            wspec(WohT.shape), wspec(WogT.shape), wspec((1, 1)),
        ],
        out_specs=pl.BlockSpec((1, TC_BLK), lambda i: (0, i)),
        out_shape=jax.ShapeDtypeStruct((1, B), jnp.float32),
    )(guT, giT, muT, miT,
      W0uT, W0iT, b0.reshape(-1, 1), W1.T, b1.reshape(-1, 1),
      W2.T, b2.reshape(-1, 1), W3.T, b3.reshape(-1, 1),
      WohT, WogT, b_out.reshape(1, 1))
    return predT.reshape(B, 1)
